# baseline (speedup 1.0000x reference)
import functools

import jax
import jax.numpy as jnp
from jax import lax
from jax.experimental import pallas as pl
from jax.experimental.pallas import tpu as pltpu
from jax.experimental.pallas import tpu_sc as plsc

_NUM_E = 8192
_DIM = 256
_BM = 256
_COMMIT = 0.25

_ROUND_AFTER = (0, 2)


def _argmin_body(x_ref, wt_ref, x2_ref, w2_ref, idx_ref):
    x = x_ref[...]
    mm = jnp.dot(x, wt_ref[...], preferred_element_type=jnp.float32)
    d = (x2_ref[...] + w2_ref[...]) - 2.0 * mm
    iota = jax.lax.broadcasted_iota(jnp.int32, d.shape, 1)

    inf = jnp.float32(jnp.inf)

    def _mr(a):
        return jnp.min(a, axis=1, keepdims=True)

    b0 = d[:, 2688:2816]
    ib0 = iota[:, 2688:2816]
    b2 = d[:, 5376:5504]
    ib2 = iota[:, 5376:5504]
    b0_lo = jnp.where(ib0 < 2736, b0, inf)
    b0_hi = jnp.where(ib0 >= 2736, b0, inf)
    b2_lo = jnp.where(ib2 < 5472, b2, inf)
    b2_hi = jnp.where(ib2 >= 5472, b2, inf)

    parts = [
        ((d[:, 0:2688], iota[:, 0:2688]), (b0_lo, ib0)),
        ((b0_hi, ib0), (d[:, 2816:4096], iota[:, 2816:4096])),
        ((d[:, 4096:5376], iota[:, 4096:5376]), (b2_lo, ib2)),
        ((b2_hi, ib2), (d[:, 5504:8192], iota[:, 5504:8192])),
    ]

    acc_v = None
    acc_i = None
    for s, ((da, ia), (db, ib)) in enumerate(parts):
        m_s = jnp.minimum(_mr(da), _mr(db))
        i_s = jnp.minimum(
            _mr(jnp.where(da == m_s, ia, _NUM_E)),
            _mr(jnp.where(db == m_s, ib, _NUM_E)))
        if acc_v is None:
            acc_v, acc_i = m_s, i_s
        else:
            repl = m_s < acc_v
            tie = m_s == acc_v
            acc_i = jnp.where(repl | (tie & (i_s < acc_i)), i_s, acc_i)
            acc_v = jnp.where(repl | tie, m_s, acc_v)
        if s in _ROUND_AFTER:
            acc_v = acc_v.astype(jnp.bfloat16).astype(jnp.float32)

    idx_ref[...] = acc_i


def _onehot_body(idx_ref, enc_ref):
    iota = jax.lax.broadcasted_iota(jnp.int32, enc_ref.shape, 1)
    enc_ref[...] = (iota == idx_ref[...]).astype(jnp.float32)


def _ste_body(x_ref, q_ref, ste_ref, acc_ref):
    i = pl.program_id(0)
    x = x_ref[...]
    t = q_ref[...] - x
    ste_ref[...] = x + t

    @pl.when(i == 0)
    def _():
        acc_ref[...] = jnp.zeros((1, 1), jnp.float32)
    acc_ref[...] += jnp.sum(t * t, axis=(0, 1), keepdims=True)


def _perp_body(idx_ref, out_ref):
    idx = idx_ref[...]
    b_count = idx.shape[0]
    counts = jnp.zeros(idx.shape, jnp.int32)
    for b in range(b_count):
        counts += (idx == idx[b:b + 1, :]).astype(jnp.int32)
    p = counts.astype(jnp.float32) * (1.0 / b_count)
    s = jnp.sum(jnp.log(p + 1e-05), axis=(0, 1), keepdims=True) * (1.0 / b_count)
    out_ref[...] = jnp.exp(-s)


def _sc_gather(table, idx):
    rows = idx.shape[0]
    info = plsc.get_sparse_core_info()
    nc = info.num_cores
    nw = nc * info.num_subcores
    b_per_w = rows // nw
    mesh = plsc.VectorSubcoreMesh(core_axis_name="c", subcore_axis_name="s")

    @functools.partial(
        pl.kernel,
        mesh=mesh,
        out_type=jax.ShapeDtypeStruct((rows, _DIM), jnp.float32),
        scratch_types=[
            pltpu.VMEM((b_per_w,), jnp.int32),
            pltpu.VMEM((b_per_w, _DIM), jnp.float32),
            pltpu.SemaphoreType.DMA,
        ],
    )
    def k(table_hbm, idx_hbm, out_hbm, idx_v, rows_v, sem):
        wid = lax.axis_index("s") * nc + lax.axis_index("c")
        base = wid * b_per_w
        pltpu.sync_copy(idx_hbm.at[pl.ds(base, b_per_w)], idx_v)
        pltpu.async_copy(table_hbm.at[idx_v], rows_v, sem).wait()
        pltpu.sync_copy(rows_v, out_hbm.at[pl.ds(base, b_per_w)])

    return k(table, idx)


def kernel(inputs, weight):
    b_count = inputs.shape[0]
    x = inputs.reshape(-1, _DIM)
    rows = x.shape[0]
    t_count = rows // b_count
    x2 = jnp.sum(x ** 2, axis=1, keepdims=True)
    w2 = jnp.sum(weight ** 2, axis=1)
    wt = weight.T

    idxo = pl.pallas_call(
        _argmin_body,
        grid=(rows // _BM,),
        in_specs=[
            pl.BlockSpec((_BM, _DIM), lambda i: (i, 0)),
            pl.BlockSpec((_DIM, _NUM_E), lambda i: (0, 0)),
            pl.BlockSpec((_BM, 1), lambda i: (i, 0)),
            pl.BlockSpec((1, _NUM_E), lambda i: (0, 0)),
        ],
        out_specs=pl.BlockSpec((_BM, 1), lambda i: (i, 0)),
        out_shape=jax.ShapeDtypeStruct((rows, 1), jnp.int32),
    )(x, wt, x2, jnp.reshape(w2, (1, _NUM_E)))

    q = _sc_gather(weight, idxo.reshape(rows))

    enc = pl.pallas_call(
        _onehot_body,
        grid=(rows // _BM,),
        in_specs=[pl.BlockSpec((_BM, 1), lambda i: (i, 0))],
        out_specs=pl.BlockSpec((_BM, _NUM_E), lambda i: (i, 0)),
        out_shape=jax.ShapeDtypeStruct((rows, _NUM_E), jnp.float32),
    )(idxo)

    ste, acc = pl.pallas_call(
        _ste_body,
        grid=(rows // _BM,),
        in_specs=[
            pl.BlockSpec((_BM, _DIM), lambda i: (i, 0)),
            pl.BlockSpec((_BM, _DIM), lambda i: (i, 0)),
        ],
        out_specs=[
            pl.BlockSpec((_BM, _DIM), lambda i: (i, 0)),
            pl.BlockSpec((1, 1), lambda i: (0, 0)),
        ],
        out_shape=[
            jax.ShapeDtypeStruct((rows, _DIM), jnp.float32),
            jax.ShapeDtypeStruct((1, 1), jnp.float32),
        ],
        compiler_params=pltpu.CompilerParams(
            dimension_semantics=("arbitrary",)),
    )(x, q)

    encoding_indices = idxo.reshape(b_count, t_count)
    encodings = enc.reshape(b_count, t_count, _NUM_E)
    quantized_ste = ste.reshape(b_count, -1)

    m = acc[0, 0] / (rows * _DIM)
    loss = m + _COMMIT * m

    perp = pl.pallas_call(
        _perp_body,
        out_shape=jax.ShapeDtypeStruct((1, 1), jnp.float32),
    )(encoding_indices)[0, 0]

    return (loss, quantized_ste, perp, encoding_indices, encodings)


# speedup vs baseline: 1.2455x; 1.2455x over previous
import functools

import jax
import jax.numpy as jnp
from jax import lax
from jax.experimental import pallas as pl
from jax.experimental.pallas import tpu as pltpu
from jax.experimental.pallas import tpu_sc as plsc

_NUM_E = 8192
_DIM = 256
_BM = 256
_COMMIT = 0.25

_ROUND_AFTER = (0, 2)


def _argmin_body(x_ref, wt_ref, x2_ref, w2_ref, idx_ref, enc_ref):
    x = x_ref[...]
    mm = jnp.dot(x, wt_ref[...], preferred_element_type=jnp.float32)
    d = (x2_ref[...] + w2_ref[...]) - 2.0 * mm
    iota = jax.lax.broadcasted_iota(jnp.int32, d.shape, 1)

    inf = jnp.float32(jnp.inf)

    def _mr(a):
        return jnp.min(a, axis=1, keepdims=True)

    b0 = d[:, 2688:2816]
    ib0 = iota[:, 2688:2816]
    b2 = d[:, 5376:5504]
    ib2 = iota[:, 5376:5504]
    b0_lo = jnp.where(ib0 < 2736, b0, inf)
    b0_hi = jnp.where(ib0 >= 2736, b0, inf)
    b2_lo = jnp.where(ib2 < 5472, b2, inf)
    b2_hi = jnp.where(ib2 >= 5472, b2, inf)

    parts = [
        ((d[:, 0:2688], iota[:, 0:2688]), (b0_lo, ib0)),
        ((b0_hi, ib0), (d[:, 2816:4096], iota[:, 2816:4096])),
        ((d[:, 4096:5376], iota[:, 4096:5376]), (b2_lo, ib2)),
        ((b2_hi, ib2), (d[:, 5504:8192], iota[:, 5504:8192])),
    ]

    acc_v = None
    acc_i = None
    for s, ((da, ia), (db, ib)) in enumerate(parts):
        m_s = jnp.minimum(_mr(da), _mr(db))
        i_s = jnp.minimum(
            _mr(jnp.where(da == m_s, ia, _NUM_E)),
            _mr(jnp.where(db == m_s, ib, _NUM_E)))
        if acc_v is None:
            acc_v, acc_i = m_s, i_s
        else:
            repl = m_s < acc_v
            tie = m_s == acc_v
            acc_i = jnp.where(repl | (tie & (i_s < acc_i)), i_s, acc_i)
            acc_v = jnp.where(repl | tie, m_s, acc_v)
        if s in _ROUND_AFTER:
            acc_v = acc_v.astype(jnp.bfloat16).astype(jnp.float32)

    idx_ref[...] = acc_i
    enc_ref[...] = (iota == acc_i).astype(jnp.float32)


def _ste_body(x_ref, q_ref, ste_ref, acc_ref):
    i = pl.program_id(0)
    x = x_ref[...]
    t = q_ref[...] - x
    ste_ref[...] = x + t

    @pl.when(i == 0)
    def _():
        acc_ref[...] = jnp.zeros((1, 1), jnp.float32)
    acc_ref[...] += jnp.sum(t * t, axis=(0, 1), keepdims=True)


def _perp_body(idx_ref, out_ref):
    idx = idx_ref[...]
    b_count = idx.shape[0]
    counts = jnp.zeros(idx.shape, jnp.int32)
    for b in range(b_count):
        counts += (idx == idx[b:b + 1, :]).astype(jnp.int32)
    p = counts.astype(jnp.float32) * (1.0 / b_count)
    s = jnp.sum(jnp.log(p + 1e-05), axis=(0, 1), keepdims=True) * (1.0 / b_count)
    out_ref[...] = jnp.exp(-s)


def _sc_gather(table, idx):
    rows = idx.shape[0]
    info = plsc.get_sparse_core_info()
    nc = info.num_cores
    nw = nc * info.num_subcores
    b_per_w = rows // nw
    mesh = plsc.VectorSubcoreMesh(core_axis_name="c", subcore_axis_name="s")

    @functools.partial(
        pl.kernel,
        mesh=mesh,
        out_type=jax.ShapeDtypeStruct((rows, _DIM), jnp.float32),
        scratch_types=[
            pltpu.VMEM((b_per_w,), jnp.int32),
            pltpu.VMEM((b_per_w, _DIM), jnp.float32),
            pltpu.SemaphoreType.DMA,
        ],
    )
    def k(table_hbm, idx_hbm, out_hbm, idx_v, rows_v, sem):
        wid = lax.axis_index("s") * nc + lax.axis_index("c")
        base = wid * b_per_w
        pltpu.sync_copy(idx_hbm.at[pl.ds(base, b_per_w)], idx_v)
        pltpu.async_copy(table_hbm.at[idx_v], rows_v, sem).wait()
        pltpu.sync_copy(rows_v, out_hbm.at[pl.ds(base, b_per_w)])

    return k(table, idx)


def kernel(inputs, weight):
    b_count = inputs.shape[0]
    x = inputs.reshape(-1, _DIM)
    rows = x.shape[0]
    t_count = rows // b_count
    x2 = jnp.sum(x ** 2, axis=1, keepdims=True)
    w2 = jnp.sum(weight ** 2, axis=1)
    wt = weight.T

    idxo, enc = pl.pallas_call(
        _argmin_body,
        grid=(rows // _BM,),
        in_specs=[
            pl.BlockSpec((_BM, _DIM), lambda i: (i, 0)),
            pl.BlockSpec((_DIM, _NUM_E), lambda i: (0, 0)),
            pl.BlockSpec((_BM, 1), lambda i: (i, 0)),
            pl.BlockSpec((1, _NUM_E), lambda i: (0, 0)),
        ],
        out_specs=[
            pl.BlockSpec((_BM, 1), lambda i: (i, 0)),
            pl.BlockSpec((_BM, _NUM_E), lambda i: (i, 0)),
        ],
        out_shape=[
            jax.ShapeDtypeStruct((rows, 1), jnp.int32),
            jax.ShapeDtypeStruct((rows, _NUM_E), jnp.float32),
        ],
    )(x, wt, x2, jnp.reshape(w2, (1, _NUM_E)))

    q = _sc_gather(weight, idxo.reshape(rows))

    ste, acc = pl.pallas_call(
        _ste_body,
        grid=(rows // _BM,),
        in_specs=[
            pl.BlockSpec((_BM, _DIM), lambda i: (i, 0)),
            pl.BlockSpec((_BM, _DIM), lambda i: (i, 0)),
        ],
        out_specs=[
            pl.BlockSpec((_BM, _DIM), lambda i: (i, 0)),
            pl.BlockSpec((1, 1), lambda i: (0, 0)),
        ],
        out_shape=[
            jax.ShapeDtypeStruct((rows, _DIM), jnp.float32),
            jax.ShapeDtypeStruct((1, 1), jnp.float32),
        ],
        compiler_params=pltpu.CompilerParams(
            dimension_semantics=("arbitrary",)),
    )(x, q)

    encoding_indices = idxo.reshape(b_count, t_count)
    encodings = enc.reshape(b_count, t_count, _NUM_E)
    quantized_ste = ste.reshape(b_count, -1)

    m = acc[0, 0] / (rows * _DIM)
    loss = m + _COMMIT * m

    perp = pl.pallas_call(
        _perp_body,
        out_shape=jax.ShapeDtypeStruct((1, 1), jnp.float32),
    )(encoding_indices)[0, 0]

    return (loss, quantized_ste, perp, encoding_indices, encodings)


# speedup vs baseline: 1.3088x; 1.0508x over previous
import functools

import jax
import jax.numpy as jnp
from jax import lax
from jax.experimental import pallas as pl
from jax.experimental.pallas import tpu as pltpu
from jax.experimental.pallas import tpu_sc as plsc

_NUM_E = 8192
_DIM = 256
_BM = 512
_COMMIT = 0.25

_ROUND_AFTER = (0, 2)


def _argmin_body(x_ref, wt_ref, x2_ref, w2_ref, idx_ref, enc_ref):
    x = x_ref[...]
    mm = jnp.dot(x, wt_ref[...], preferred_element_type=jnp.float32)
    d = (x2_ref[...] + w2_ref[...]) - 2.0 * mm
    iota = jax.lax.broadcasted_iota(jnp.int32, d.shape, 1)

    inf = jnp.float32(jnp.inf)

    def _mr(a):
        return jnp.min(a, axis=1, keepdims=True)

    b0 = d[:, 2688:2816]
    ib0 = iota[:, 2688:2816]
    b2 = d[:, 5376:5504]
    ib2 = iota[:, 5376:5504]
    b0_lo = jnp.where(ib0 < 2736, b0, inf)
    b0_hi = jnp.where(ib0 >= 2736, b0, inf)
    b2_lo = jnp.where(ib2 < 5472, b2, inf)
    b2_hi = jnp.where(ib2 >= 5472, b2, inf)

    parts = [
        ((d[:, 0:2688], iota[:, 0:2688]), (b0_lo, ib0)),
        ((b0_hi, ib0), (d[:, 2816:4096], iota[:, 2816:4096])),
        ((d[:, 4096:5376], iota[:, 4096:5376]), (b2_lo, ib2)),
        ((b2_hi, ib2), (d[:, 5504:8192], iota[:, 5504:8192])),
    ]

    acc_v = None
    acc_i = None
    for s, ((da, ia), (db, ib)) in enumerate(parts):
        m_s = jnp.minimum(_mr(da), _mr(db))
        i_s = jnp.minimum(
            _mr(jnp.where(da == m_s, ia, _NUM_E)),
            _mr(jnp.where(db == m_s, ib, _NUM_E)))
        if acc_v is None:
            acc_v, acc_i = m_s, i_s
        else:
            repl = m_s < acc_v
            tie = m_s == acc_v
            acc_i = jnp.where(repl | (tie & (i_s < acc_i)), i_s, acc_i)
            acc_v = jnp.where(repl | tie, m_s, acc_v)
        if s in _ROUND_AFTER:
            acc_v = acc_v.astype(jnp.bfloat16).astype(jnp.float32)

    idx_ref[...] = acc_i
    enc_ref[...] = (iota == acc_i).astype(jnp.float32)


def _ste_body(x_ref, q_ref, ste_ref, acc_ref):
    i = pl.program_id(0)
    x = x_ref[...]
    t = q_ref[...] - x
    ste_ref[...] = x + t

    @pl.when(i == 0)
    def _():
        acc_ref[...] = jnp.zeros((1, 1), jnp.float32)
    acc_ref[...] += jnp.sum(t * t, axis=(0, 1), keepdims=True)


def _perp_body(idx_ref, out_ref):
    idx = idx_ref[...]
    b_count = idx.shape[0]
    counts = jnp.zeros(idx.shape, jnp.int32)
    for b in range(b_count):
        counts += (idx == idx[b:b + 1, :]).astype(jnp.int32)
    p = counts.astype(jnp.float32) * (1.0 / b_count)
    s = jnp.sum(jnp.log(p + 1e-05), axis=(0, 1), keepdims=True) * (1.0 / b_count)
    out_ref[...] = jnp.exp(-s)


def _sc_gather(table, idx):
    rows = idx.shape[0]
    info = plsc.get_sparse_core_info()
    nc = info.num_cores
    nw = nc * info.num_subcores
    b_per_w = rows // nw
    mesh = plsc.VectorSubcoreMesh(core_axis_name="c", subcore_axis_name="s")

    @functools.partial(
        pl.kernel,
        mesh=mesh,
        out_type=jax.ShapeDtypeStruct((rows, _DIM), jnp.float32),
        scratch_types=[
            pltpu.VMEM((b_per_w,), jnp.int32),
            pltpu.VMEM((b_per_w, _DIM), jnp.float32),
            pltpu.SemaphoreType.DMA,
        ],
    )
    def k(table_hbm, idx_hbm, out_hbm, idx_v, rows_v, sem):
        wid = lax.axis_index("s") * nc + lax.axis_index("c")
        base = wid * b_per_w
        pltpu.sync_copy(idx_hbm.at[pl.ds(base, b_per_w)], idx_v)
        pltpu.async_copy(table_hbm.at[idx_v], rows_v, sem).wait()
        pltpu.sync_copy(rows_v, out_hbm.at[pl.ds(base, b_per_w)])

    return k(table, idx)


def kernel(inputs, weight):
    b_count = inputs.shape[0]
    x = inputs.reshape(-1, _DIM)
    rows = x.shape[0]
    t_count = rows // b_count
    x2 = jnp.sum(x ** 2, axis=1, keepdims=True)
    w2 = jnp.sum(weight ** 2, axis=1)
    wt = weight.T

    idxo, enc = pl.pallas_call(
        _argmin_body,
        grid=(rows // _BM,),
        in_specs=[
            pl.BlockSpec((_BM, _DIM), lambda i: (i, 0)),
            pl.BlockSpec((_DIM, _NUM_E), lambda i: (0, 0)),
            pl.BlockSpec((_BM, 1), lambda i: (i, 0)),
            pl.BlockSpec((1, _NUM_E), lambda i: (0, 0)),
        ],
        out_specs=[
            pl.BlockSpec((_BM, 1), lambda i: (i, 0)),
            pl.BlockSpec((_BM, _NUM_E), lambda i: (i, 0)),
        ],
        out_shape=[
            jax.ShapeDtypeStruct((rows, 1), jnp.int32),
            jax.ShapeDtypeStruct((rows, _NUM_E), jnp.float32),
        ],
    )(x, wt, x2, jnp.reshape(w2, (1, _NUM_E)))

    q = _sc_gather(weight, idxo.reshape(rows))

    ste, acc = pl.pallas_call(
        _ste_body,
        grid=(rows // _BM,),
        in_specs=[
            pl.BlockSpec((_BM, _DIM), lambda i: (i, 0)),
            pl.BlockSpec((_BM, _DIM), lambda i: (i, 0)),
        ],
        out_specs=[
            pl.BlockSpec((_BM, _DIM), lambda i: (i, 0)),
            pl.BlockSpec((1, 1), lambda i: (0, 0)),
        ],
        out_shape=[
            jax.ShapeDtypeStruct((rows, _DIM), jnp.float32),
            jax.ShapeDtypeStruct((1, 1), jnp.float32),
        ],
        compiler_params=pltpu.CompilerParams(
            dimension_semantics=("arbitrary",)),
    )(x, q)

    encoding_indices = idxo.reshape(b_count, t_count)
    encodings = enc.reshape(b_count, t_count, _NUM_E)
    quantized_ste = ste.reshape(b_count, -1)

    m = acc[0, 0] / (rows * _DIM)
    loss = m + _COMMIT * m

    perp = pl.pallas_call(
        _perp_body,
        out_shape=jax.ShapeDtypeStruct((1, 1), jnp.float32),
    )(encoding_indices)[0, 0]

    return (loss, quantized_ste, perp, encoding_indices, encodings)
